# baseline (speedup 1.0000x reference)
import functools

import jax
import jax.numpy as jnp
from jax import lax
from jax.experimental import pallas as pl
from jax.experimental.pallas import tpu as pltpu, tpu_sc as plsc

_B, _N, _M = 8, 4096, 1024
_C1, _C2 = 128, 256
_H1, _H2 = 256, 128
_BN = 512
_NB = _N // _BN

_NPTS = _B * _N
_NROWS = _B * _M
_NW = 32
_PPW = _NPTS // _NW
_CHUNK = 32
_NCH = _PPW // _CHUNK
_IDXC = 3 * _CHUNK
_WEXPC = _IDXC * 16

_interpret = False


def _dot(a, b):
    return jax.lax.dot_general(
        a, b, (((1,), (0,)), ((), ())),
        preferred_element_type=jnp.float32)


def _k1_body(xyz1_ref, xyz2t_ref, idx_ref, wexp_ref):
    b = pl.program_id(0)

    x1 = xyz1_ref[0]
    x2t = xyz2t_ref[0]

    mm = _dot(x1, x2t)
    s1 = x1[:, 0:1] * x1[:, 0:1] + x1[:, 1:2] * x1[:, 1:2] + x1[:, 2:3] * x1[:, 2:3]
    s2 = x2t[0:1, :] * x2t[0:1, :] + x2t[1:2, :] * x2t[1:2, :] + x2t[2:3, :] * x2t[2:3, :]
    d = -2.0 * mm
    d = d + s1
    d = d + s2

    iota = jax.lax.broadcasted_iota(jnp.int32, (_BN, _M), 1)
    big = jnp.float32(jnp.inf)
    recips = []
    cols = []
    for _ in range(3):
        m = jnp.min(d, axis=1, keepdims=True)
        col = jnp.min(jnp.where(d == m, iota, _M), axis=1, keepdims=True)
        mask = iota == col
        d = jnp.where(mask, big, d)
        recips.append(1.0 / (m + 1e-8))
        cols.append(col + b * _M)
    norm = recips[0] + recips[1] + recips[2]

    idx_ref[0] = jnp.concatenate(cols, axis=1)
    ws = [jnp.broadcast_to(recips[k] / norm, (_BN, 16)) for k in range(3)]
    wexp_ref[0] = jnp.concatenate(ws, axis=1)


_sc_mesh = plsc.VectorSubcoreMesh(core_axis_name="c", subcore_axis_name="s")


@functools.partial(
    pl.kernel,
    out_type=jax.ShapeDtypeStruct((_NPTS * _C2,), jnp.float32),
    mesh=_sc_mesh,
    scratch_types=[
        pltpu.VMEM((_IDXC,), jnp.int32),
        pltpu.VMEM((_IDXC, _C2), jnp.float32),
        pltpu.VMEM((_CHUNK * _C2,), jnp.float32),
        pltpu.VMEM((_WEXPC,), jnp.float32),
        pltpu.SemaphoreType.DMA,
    ],
)
def _sc_interp(p2_hbm, gidx_hbm, wexp_hbm, out_hbm,
               idx_v, rows_v, out_v, w_v, sem):
    wid = lax.axis_index("s") * 2 + lax.axis_index("c")
    base_i = wid * (_PPW * 3)

    def chunk_body(c, _):
        off = base_i + c * _IDXC
        pltpu.sync_copy(gidx_hbm.at[pl.ds(off, _IDXC)], idx_v)
        pltpu.sync_copy(wexp_hbm.at[pl.ds(off * 16, _WEXPC)], w_v)
        pltpu.async_copy(p2_hbm.at[idx_v], rows_v, sem).wait()

        def point_body(i, _):
            w0 = w_v[pl.ds((3 * i) * 16, 16)]
            w1 = w_v[pl.ds((3 * i + 1) * 16, 16)]
            w2 = w_v[pl.ds((3 * i + 2) * 16, 16)]
            for c16 in range(_C2 // 16):
                o = c16 * 16
                r0 = rows_v[3 * i, pl.ds(o, 16)]
                r1 = rows_v[3 * i + 1, pl.ds(o, 16)]
                r2 = rows_v[3 * i + 2, pl.ds(o, 16)]
                acc = r0 * w0
                acc = acc + r1 * w1
                acc = acc + r2 * w2
                out_v[pl.ds(i * _C2 + o, 16)] = acc
            return 0

        lax.fori_loop(0, _CHUNK, point_body, 0)
        dst = (wid * _PPW + c * _CHUNK) * _C2
        pltpu.sync_copy(out_v, out_hbm.at[pl.ds(dst, _CHUNK * _C2)])
        return 0

    lax.fori_loop(0, _NCH, chunk_body, 0)


def _k2_body(p1_ref, interp_ref, w1t_ref, b1_ref, h1_ref, stats_ref):
    b = pl.program_id(0)
    nb = pl.program_id(1)
    newp = jnp.concatenate([p1_ref[0], interp_ref[0]], axis=1)
    h1 = _dot(newp, w1t_ref[...]) + b1_ref[...]
    h1_ref[0] = h1

    @pl.when((b == 0) & (nb == 0))
    def _():
        stats_ref[...] = jnp.zeros((8, _H1), jnp.float32)

    stats_ref[0:1, :] += jnp.sum(h1, axis=0, keepdims=True)
    stats_ref[1:2, :] += jnp.sum(h1 * h1, axis=0, keepdims=True)


def _bn_scale_shift(stats, g, beta, h_dim):
    cnt = jnp.float32(_B * _N)
    mean = stats[0:1, :] / cnt
    var = stats[1:2, :] / cnt - mean * mean
    rstd = jax.lax.rsqrt(var + 1e-5)
    scale = rstd * g
    shift = beta - mean * scale
    return scale, shift


def _k3_body(h1_ref, stats1_ref, g1_ref, beta1_ref, w2t_ref, b2_ref,
             h2_ref, stats_ref):
    b = pl.program_id(0)
    nb = pl.program_id(1)
    scale, shift = _bn_scale_shift(stats1_ref[...], g1_ref[...], beta1_ref[...], _H1)
    hn = jnp.maximum(h1_ref[0] * scale + shift, 0.0)
    h2 = _dot(hn, w2t_ref[...]) + b2_ref[...]
    h2_ref[0] = h2

    @pl.when((b == 0) & (nb == 0))
    def _():
        stats_ref[...] = jnp.zeros((8, _H2), jnp.float32)

    stats_ref[0:1, :] += jnp.sum(h2, axis=0, keepdims=True)
    stats_ref[1:2, :] += jnp.sum(h2 * h2, axis=0, keepdims=True)


def _k4_body(h2_ref, stats2_ref, g2_ref, beta2_ref, out_ref):
    scale, shift = _bn_scale_shift(stats2_ref[...], g2_ref[...], beta2_ref[...], _H2)
    out_ref[0] = jnp.maximum(h2_ref[0] * scale + shift, 0.0)


def kernel(xyz1, xyz2, points1, points2, W1, b1, g1, beta1, W2, b2, g2, beta2):
    xyz2t = jnp.transpose(xyz2, (0, 2, 1))
    w1t = jnp.transpose(W1)
    w2t = jnp.transpose(W2)
    b1r = b1.reshape(1, _H1)
    b2r = b2.reshape(1, _H2)
    g1r = g1.reshape(1, _H1)
    beta1r = beta1.reshape(1, _H1)
    g2r = g2.reshape(1, _H2)
    beta2r = beta2.reshape(1, _H2)

    grid = (_B, _NB)

    gidx, wexp = pl.pallas_call(
        _k1_body,
        grid=grid,
        in_specs=[
            pl.BlockSpec((1, _BN, 3), lambda b, n: (b, n, 0)),
            pl.BlockSpec((1, 3, _M), lambda b, n: (b, 0, 0)),
        ],
        out_specs=[
            pl.BlockSpec((1, _BN, 3), lambda b, n: (b, n, 0)),
            pl.BlockSpec((1, _BN, 48), lambda b, n: (b, n, 0)),
        ],
        out_shape=[
            jax.ShapeDtypeStruct((_B, _N, 3), jnp.int32),
            jax.ShapeDtypeStruct((_B, _N, 48), jnp.float32),
        ],
        interpret=_interpret,
    )(xyz1, xyz2t)

    interp_flat = _sc_interp(points2.reshape(_NROWS, _C2),
                             gidx.reshape(_NPTS * 3),
                             wexp.reshape(_NPTS * 48))
    interp = interp_flat.reshape(_B, _N, _C2)

    h1, stats1 = pl.pallas_call(
        _k2_body,
        grid=grid,
        in_specs=[
            pl.BlockSpec((1, _BN, _C1), lambda b, n: (b, n, 0)),
            pl.BlockSpec((1, _BN, _C2), lambda b, n: (b, n, 0)),
            pl.BlockSpec((_C1 + _C2, _H1), lambda b, n: (0, 0)),
            pl.BlockSpec((1, _H1), lambda b, n: (0, 0)),
        ],
        out_specs=[
            pl.BlockSpec((1, _BN, _H1), lambda b, n: (b, n, 0)),
            pl.BlockSpec((8, _H1), lambda b, n: (0, 0)),
        ],
        out_shape=[
            jax.ShapeDtypeStruct((_B, _N, _H1), jnp.float32),
            jax.ShapeDtypeStruct((8, _H1), jnp.float32),
        ],
        interpret=_interpret,
    )(points1, interp, w1t, b1r)

    h2, stats2 = pl.pallas_call(
        _k3_body,
        grid=grid,
        in_specs=[
            pl.BlockSpec((1, _BN, _H1), lambda b, n: (b, n, 0)),
            pl.BlockSpec((8, _H1), lambda b, n: (0, 0)),
            pl.BlockSpec((1, _H1), lambda b, n: (0, 0)),
            pl.BlockSpec((1, _H1), lambda b, n: (0, 0)),
            pl.BlockSpec((_H1, _H2), lambda b, n: (0, 0)),
            pl.BlockSpec((1, _H2), lambda b, n: (0, 0)),
        ],
        out_specs=[
            pl.BlockSpec((1, _BN, _H2), lambda b, n: (b, n, 0)),
            pl.BlockSpec((8, _H2), lambda b, n: (0, 0)),
        ],
        out_shape=[
            jax.ShapeDtypeStruct((_B, _N, _H2), jnp.float32),
            jax.ShapeDtypeStruct((8, _H2), jnp.float32),
        ],
        interpret=_interpret,
    )(h1, stats1, g1r, beta1r, w2t, b2r)

    out = pl.pallas_call(
        _k4_body,
        grid=grid,
        in_specs=[
            pl.BlockSpec((1, _BN, _H2), lambda b, n: (b, n, 0)),
            pl.BlockSpec((8, _H2), lambda b, n: (0, 0)),
            pl.BlockSpec((1, _H2), lambda b, n: (0, 0)),
            pl.BlockSpec((1, _H2), lambda b, n: (0, 0)),
        ],
        out_specs=pl.BlockSpec((1, _BN, _H2), lambda b, n: (b, n, 0)),
        out_shape=jax.ShapeDtypeStruct((_B, _N, _H2), jnp.float32),
        interpret=_interpret,
    )(h2, stats2, g2r, beta2r)

    return out


# speedup vs baseline: 1.1595x; 1.1595x over previous
import functools

import jax
import jax.numpy as jnp
from jax import lax
from jax.experimental import pallas as pl
from jax.experimental.pallas import tpu as pltpu, tpu_sc as plsc

_B, _N, _M = 8, 4096, 1024
_C1, _C2 = 128, 256
_H1, _H2 = 256, 128
_BN = 512
_NB = _N // _BN

_NPTS = _B * _N
_NROWS = _B * _M
_NW = 32
_PPW = _NPTS // _NW
_CHUNK = 32
_NCH = _PPW // _CHUNK
_IDXC = 3 * _CHUNK
_WEXPC = _IDXC * 16

_interpret = False


def _dot(a, b):
    return jax.lax.dot_general(
        a, b, (((1,), (0,)), ((), ())),
        preferred_element_type=jnp.float32)


def _k1_body(xyz1_ref, xyz2t_ref, idx_ref, wexp_ref):
    b = pl.program_id(0)

    x1 = xyz1_ref[0]
    x2t = xyz2t_ref[0]

    mm = _dot(x1, x2t)
    s1 = x1[:, 0:1] * x1[:, 0:1] + x1[:, 1:2] * x1[:, 1:2] + x1[:, 2:3] * x1[:, 2:3]
    s2 = x2t[0:1, :] * x2t[0:1, :] + x2t[1:2, :] * x2t[1:2, :] + x2t[2:3, :] * x2t[2:3, :]
    d = -2.0 * mm
    d = d + s1
    d = d + s2

    iota = jax.lax.broadcasted_iota(jnp.int32, (_BN, _M), 1)
    big = jnp.float32(jnp.inf)
    recips = []
    cols = []
    for _ in range(3):
        m = jnp.min(d, axis=1, keepdims=True)
        col = jnp.min(jnp.where(d == m, iota, _M), axis=1, keepdims=True)
        mask = iota == col
        d = jnp.where(mask, big, d)
        recips.append(1.0 / (m + 1e-8))
        cols.append(col + b * _M)
    norm = recips[0] + recips[1] + recips[2]

    idx_ref[0] = jnp.concatenate(cols, axis=1)
    ws = [jnp.broadcast_to(recips[k] / norm, (_BN, 16)) for k in range(3)]
    wexp_ref[0] = jnp.concatenate(ws, axis=1)


_sc_mesh = plsc.VectorSubcoreMesh(core_axis_name="c", subcore_axis_name="s")


@functools.partial(
    pl.kernel,
    out_type=jax.ShapeDtypeStruct((_NPTS * _C2,), jnp.float32),
    mesh=_sc_mesh,
    scratch_types=[
        pltpu.VMEM((3 * _PPW,), jnp.int32),
        pltpu.VMEM((_IDXC, _C2), jnp.float32),
        pltpu.VMEM((_IDXC, _C2), jnp.float32),
        pltpu.VMEM((_WEXPC,), jnp.float32),
        pltpu.VMEM((_WEXPC,), jnp.float32),
        pltpu.VMEM((_CHUNK * _C2,), jnp.float32),
        pltpu.SemaphoreType.DMA,
        pltpu.SemaphoreType.DMA,
        pltpu.SemaphoreType.DMA,
        pltpu.SemaphoreType.DMA,
    ],
)
def _sc_interp(p2_hbm, gidx_hbm, wexp_hbm, out_hbm,
               idx_all, rows0, rows1, w0, w1, out_v,
               sg0, sg1, sw0, sw1):
    wid = lax.axis_index("s") * 2 + lax.axis_index("c")
    base_i = wid * (_PPW * 3)

    pltpu.sync_copy(gidx_hbm.at[pl.ds(base_i, _PPW * 3)], idx_all)

    def start_chunk(c, rows_buf, w_buf, sg, sw):
        pltpu.async_copy(
            p2_hbm.at[idx_all.at[pl.ds(c * _IDXC, _IDXC)]], rows_buf, sg)
        pltpu.async_copy(
            wexp_hbm.at[pl.ds((base_i + c * _IDXC) * 16, _WEXPC)], w_buf, sw)

    def wait_chunk(rows_buf, w_buf, sg, sw):
        pltpu.make_async_copy(p2_hbm.at[pl.ds(0, _IDXC)], rows_buf, sg).wait()
        pltpu.make_async_copy(wexp_hbm.at[pl.ds(0, _WEXPC)], w_buf, sw).wait()

    def compute_chunk(c, rows_v, w_v):
        def point_body(i, _):
            pw0 = w_v[pl.ds((3 * i) * 16, 16)]
            pw1 = w_v[pl.ds((3 * i + 1) * 16, 16)]
            pw2 = w_v[pl.ds((3 * i + 2) * 16, 16)]
            for c16 in range(_C2 // 16):
                o = c16 * 16
                r0 = rows_v[3 * i, pl.ds(o, 16)]
                r1 = rows_v[3 * i + 1, pl.ds(o, 16)]
                r2 = rows_v[3 * i + 2, pl.ds(o, 16)]
                acc = r0 * pw0
                acc = acc + r1 * pw1
                acc = acc + r2 * pw2
                out_v[pl.ds(i * _C2 + o, 16)] = acc
            return 0

        lax.fori_loop(0, _CHUNK, point_body, 0)
        dst = (wid * _PPW + c * _CHUNK) * _C2
        pltpu.sync_copy(out_v, out_hbm.at[pl.ds(dst, _CHUNK * _C2)])

    start_chunk(0, rows0, w0, sg0, sw0)

    def _ring_body(j, _):
        a = 2 * j
        b = a + 1
        start_chunk(b, rows1, w1, sg1, sw1)
        wait_chunk(rows0, w0, sg0, sw0)
        compute_chunk(a, rows0, w0)

        @pl.when(b + 1 < _NCH)
        def _():
            start_chunk(b + 1, rows0, w0, sg0, sw0)

        wait_chunk(rows1, w1, sg1, sw1)
        compute_chunk(b, rows1, w1)
        return 0

    lax.fori_loop(0, _NCH // 2, _ring_body, 0)


def _k2_body(p1_ref, interp_ref, w1t_ref, b1_ref, h1_ref, stats_ref):
    b = pl.program_id(0)
    nb = pl.program_id(1)
    newp = jnp.concatenate([p1_ref[0], interp_ref[0]], axis=1)
    h1 = _dot(newp, w1t_ref[...]) + b1_ref[...]
    h1_ref[0] = h1

    @pl.when((b == 0) & (nb == 0))
    def _():
        stats_ref[...] = jnp.zeros((8, _H1), jnp.float32)

    stats_ref[0:1, :] += jnp.sum(h1, axis=0, keepdims=True)
    stats_ref[1:2, :] += jnp.sum(h1 * h1, axis=0, keepdims=True)


def _bn_scale_shift(stats, g, beta, h_dim):
    cnt = jnp.float32(_B * _N)
    mean = stats[0:1, :] / cnt
    var = stats[1:2, :] / cnt - mean * mean
    rstd = jax.lax.rsqrt(var + 1e-5)
    scale = rstd * g
    shift = beta - mean * scale
    return scale, shift


def _k3_body(h1_ref, stats1_ref, g1_ref, beta1_ref, w2t_ref, b2_ref,
             h2_ref, stats_ref):
    b = pl.program_id(0)
    nb = pl.program_id(1)
    scale, shift = _bn_scale_shift(stats1_ref[...], g1_ref[...], beta1_ref[...], _H1)
    hn = jnp.maximum(h1_ref[0] * scale + shift, 0.0)
    h2 = _dot(hn, w2t_ref[...]) + b2_ref[...]
    h2_ref[0] = h2

    @pl.when((b == 0) & (nb == 0))
    def _():
        stats_ref[...] = jnp.zeros((8, _H2), jnp.float32)

    stats_ref[0:1, :] += jnp.sum(h2, axis=0, keepdims=True)
    stats_ref[1:2, :] += jnp.sum(h2 * h2, axis=0, keepdims=True)


def _k4_body(h2_ref, stats2_ref, g2_ref, beta2_ref, out_ref):
    scale, shift = _bn_scale_shift(stats2_ref[...], g2_ref[...], beta2_ref[...], _H2)
    out_ref[0] = jnp.maximum(h2_ref[0] * scale + shift, 0.0)


def kernel(xyz1, xyz2, points1, points2, W1, b1, g1, beta1, W2, b2, g2, beta2):
    xyz2t = jnp.transpose(xyz2, (0, 2, 1))
    w1t = jnp.transpose(W1)
    w2t = jnp.transpose(W2)
    b1r = b1.reshape(1, _H1)
    b2r = b2.reshape(1, _H2)
    g1r = g1.reshape(1, _H1)
    beta1r = beta1.reshape(1, _H1)
    g2r = g2.reshape(1, _H2)
    beta2r = beta2.reshape(1, _H2)

    grid = (_B, _NB)

    gidx, wexp = pl.pallas_call(
        _k1_body,
        grid=grid,
        in_specs=[
            pl.BlockSpec((1, _BN, 3), lambda b, n: (b, n, 0)),
            pl.BlockSpec((1, 3, _M), lambda b, n: (b, 0, 0)),
        ],
        out_specs=[
            pl.BlockSpec((1, _BN, 3), lambda b, n: (b, n, 0)),
            pl.BlockSpec((1, _BN, 48), lambda b, n: (b, n, 0)),
        ],
        out_shape=[
            jax.ShapeDtypeStruct((_B, _N, 3), jnp.int32),
            jax.ShapeDtypeStruct((_B, _N, 48), jnp.float32),
        ],
        interpret=_interpret,
    )(xyz1, xyz2t)

    interp_flat = _sc_interp(points2.reshape(_NROWS, _C2),
                             gidx.reshape(_NPTS * 3),
                             wexp.reshape(_NPTS * 48))
    interp = interp_flat.reshape(_B, _N, _C2)

    h1, stats1 = pl.pallas_call(
        _k2_body,
        grid=grid,
        in_specs=[
            pl.BlockSpec((1, _BN, _C1), lambda b, n: (b, n, 0)),
            pl.BlockSpec((1, _BN, _C2), lambda b, n: (b, n, 0)),
            pl.BlockSpec((_C1 + _C2, _H1), lambda b, n: (0, 0)),
            pl.BlockSpec((1, _H1), lambda b, n: (0, 0)),
        ],
        out_specs=[
            pl.BlockSpec((1, _BN, _H1), lambda b, n: (b, n, 0)),
            pl.BlockSpec((8, _H1), lambda b, n: (0, 0)),
        ],
        out_shape=[
            jax.ShapeDtypeStruct((_B, _N, _H1), jnp.float32),
            jax.ShapeDtypeStruct((8, _H1), jnp.float32),
        ],
        interpret=_interpret,
    )(points1, interp, w1t, b1r)

    h2, stats2 = pl.pallas_call(
        _k3_body,
        grid=grid,
        in_specs=[
            pl.BlockSpec((1, _BN, _H1), lambda b, n: (b, n, 0)),
            pl.BlockSpec((8, _H1), lambda b, n: (0, 0)),
            pl.BlockSpec((1, _H1), lambda b, n: (0, 0)),
            pl.BlockSpec((1, _H1), lambda b, n: (0, 0)),
            pl.BlockSpec((_H1, _H2), lambda b, n: (0, 0)),
            pl.BlockSpec((1, _H2), lambda b, n: (0, 0)),
        ],
        out_specs=[
            pl.BlockSpec((1, _BN, _H2), lambda b, n: (b, n, 0)),
            pl.BlockSpec((8, _H2), lambda b, n: (0, 0)),
        ],
        out_shape=[
            jax.ShapeDtypeStruct((_B, _N, _H2), jnp.float32),
            jax.ShapeDtypeStruct((8, _H2), jnp.float32),
        ],
        interpret=_interpret,
    )(h1, stats1, g1r, beta1r, w2t, b2r)

    out = pl.pallas_call(
        _k4_body,
        grid=grid,
        in_specs=[
            pl.BlockSpec((1, _BN, _H2), lambda b, n: (b, n, 0)),
            pl.BlockSpec((8, _H2), lambda b, n: (0, 0)),
            pl.BlockSpec((1, _H2), lambda b, n: (0, 0)),
            pl.BlockSpec((1, _H2), lambda b, n: (0, 0)),
        ],
        out_specs=pl.BlockSpec((1, _BN, _H2), lambda b, n: (b, n, 0)),
        out_shape=jax.ShapeDtypeStruct((_B, _N, _H2), jnp.float32),
        interpret=_interpret,
    )(h2, stats2, g2r, beta2r)

    return out


# speedup vs baseline: 1.2193x; 1.0516x over previous
import functools

import jax
import jax.numpy as jnp
from jax import lax
from jax.experimental import pallas as pl
from jax.experimental.pallas import tpu as pltpu, tpu_sc as plsc

_B, _N, _M = 8, 4096, 1024
_C1, _C2 = 128, 256
_H1, _H2 = 256, 128
_BN = 512
_NB = _N // _BN

_NPTS = _B * _N
_NROWS = _B * _M
_NW = 32
_PPW = _NPTS // _NW
_CHUNK = 32
_NCH = _PPW // _CHUNK
_IDXC = 3 * _CHUNK
_WEXPC = _IDXC * 16

_interpret = False


def _dot(a, b):
    return jax.lax.dot_general(
        a, b, (((1,), (0,)), ((), ())),
        preferred_element_type=jnp.float32)


def _k1_body(xyz1_ref, xyz2t_ref, idx_ref, wexp_ref):
    b = pl.program_id(0)

    x1 = xyz1_ref[0]
    x2t = xyz2t_ref[0]

    mm = _dot(x1, x2t)
    s1 = x1[:, 0:1] * x1[:, 0:1] + x1[:, 1:2] * x1[:, 1:2] + x1[:, 2:3] * x1[:, 2:3]
    s2 = x2t[0:1, :] * x2t[0:1, :] + x2t[1:2, :] * x2t[1:2, :] + x2t[2:3, :] * x2t[2:3, :]
    d = -2.0 * mm
    d = d + s1
    d = d + s2

    iotaf = jax.lax.broadcasted_iota(jnp.int32, (_BN, _M), 1).astype(jnp.float32)
    big = jnp.float32(jnp.inf)
    recips = []
    cols = []
    for _ in range(3):
        m = jnp.min(d, axis=1, keepdims=True)
        colf = jnp.min(jnp.where(d == m, iotaf, jnp.float32(_M)),
                       axis=1, keepdims=True)
        mask = iotaf == colf
        d = jnp.where(mask, big, d)
        recips.append(1.0 / (m + 1e-8))
        cols.append(colf.astype(jnp.int32) + b * _M)
    norm = recips[0] + recips[1] + recips[2]

    idx_ref[0] = jnp.concatenate(cols, axis=1)
    ws = [jnp.broadcast_to(recips[k] / norm, (_BN, 16)) for k in range(3)]
    wexp_ref[0] = jnp.concatenate(ws, axis=1)


_sc_mesh = plsc.VectorSubcoreMesh(core_axis_name="c", subcore_axis_name="s")


@functools.partial(
    pl.kernel,
    out_type=jax.ShapeDtypeStruct((_NPTS * _C2,), jnp.float32),
    mesh=_sc_mesh,
    scratch_types=[
        pltpu.VMEM((3 * _PPW,), jnp.int32),
        pltpu.VMEM((_IDXC, _C2), jnp.float32),
        pltpu.VMEM((_IDXC, _C2), jnp.float32),
        pltpu.VMEM((_WEXPC,), jnp.float32),
        pltpu.VMEM((_WEXPC,), jnp.float32),
        pltpu.VMEM((_CHUNK * _C2,), jnp.float32),
        pltpu.SemaphoreType.DMA,
        pltpu.SemaphoreType.DMA,
        pltpu.SemaphoreType.DMA,
        pltpu.SemaphoreType.DMA,
    ],
)
def _sc_interp(p2_hbm, gidx_hbm, wexp_hbm, out_hbm,
               idx_all, rows0, rows1, w0, w1, out_v,
               sg0, sg1, sw0, sw1):
    wid = lax.axis_index("s") * 2 + lax.axis_index("c")
    base_i = wid * (_PPW * 3)

    pltpu.sync_copy(gidx_hbm.at[pl.ds(base_i, _PPW * 3)], idx_all)

    def start_chunk(c, rows_buf, w_buf, sg, sw):
        pltpu.async_copy(
            p2_hbm.at[idx_all.at[pl.ds(c * _IDXC, _IDXC)]], rows_buf, sg)
        pltpu.async_copy(
            wexp_hbm.at[pl.ds((base_i + c * _IDXC) * 16, _WEXPC)], w_buf, sw)

    def wait_chunk(rows_buf, w_buf, sg, sw):
        pltpu.make_async_copy(p2_hbm.at[pl.ds(0, _IDXC)], rows_buf, sg).wait()
        pltpu.make_async_copy(wexp_hbm.at[pl.ds(0, _WEXPC)], w_buf, sw).wait()

    def compute_chunk(c, rows_v, w_v):
        def point_body(i, _):
            pw0 = w_v[pl.ds((3 * i) * 16, 16)]
            pw1 = w_v[pl.ds((3 * i + 1) * 16, 16)]
            pw2 = w_v[pl.ds((3 * i + 2) * 16, 16)]
            for c16 in range(_C2 // 16):
                o = c16 * 16
                r0 = rows_v[3 * i, pl.ds(o, 16)]
                r1 = rows_v[3 * i + 1, pl.ds(o, 16)]
                r2 = rows_v[3 * i + 2, pl.ds(o, 16)]
                acc = r0 * pw0
                acc = acc + r1 * pw1
                acc = acc + r2 * pw2
                out_v[pl.ds(i * _C2 + o, 16)] = acc
            return 0

        lax.fori_loop(0, _CHUNK, point_body, 0)
        dst = (wid * _PPW + c * _CHUNK) * _C2
        pltpu.sync_copy(out_v, out_hbm.at[pl.ds(dst, _CHUNK * _C2)])

    start_chunk(0, rows0, w0, sg0, sw0)

    def _ring_body(j, _):
        a = 2 * j
        b = a + 1
        start_chunk(b, rows1, w1, sg1, sw1)
        wait_chunk(rows0, w0, sg0, sw0)
        compute_chunk(a, rows0, w0)

        @pl.when(b + 1 < _NCH)
        def _():
            start_chunk(b + 1, rows0, w0, sg0, sw0)

        wait_chunk(rows1, w1, sg1, sw1)
        compute_chunk(b, rows1, w1)
        return 0

    lax.fori_loop(0, _NCH // 2, _ring_body, 0)


def _k2_body(p1_ref, interp_ref, w1at_ref, w1bt_ref, b1_ref, h1_ref, stats_ref):
    b = pl.program_id(0)
    nb = pl.program_id(1)
    h1 = _dot(p1_ref[0], w1at_ref[...]) + _dot(interp_ref[0], w1bt_ref[...])
    h1 = h1 + b1_ref[...]
    h1_ref[0] = h1

    @pl.when((b == 0) & (nb == 0))
    def _():
        stats_ref[...] = jnp.zeros((8, _H1), jnp.float32)

    stats_ref[0:1, :] += jnp.sum(h1, axis=0, keepdims=True)
    stats_ref[1:2, :] += jnp.sum(h1 * h1, axis=0, keepdims=True)


def _bn_scale_shift(stats, g, beta, h_dim):
    cnt = jnp.float32(_B * _N)
    mean = stats[0:1, :] / cnt
    var = stats[1:2, :] / cnt - mean * mean
    rstd = jax.lax.rsqrt(var + 1e-5)
    scale = rstd * g
    shift = beta - mean * scale
    return scale, shift


def _k3_body(h1_ref, stats1_ref, g1_ref, beta1_ref, w2t_ref, b2_ref,
             h2_ref, stats_ref):
    b = pl.program_id(0)
    nb = pl.program_id(1)
    scale, shift = _bn_scale_shift(stats1_ref[...], g1_ref[...], beta1_ref[...], _H1)
    hn = jnp.maximum(h1_ref[0] * scale + shift, 0.0)
    h2 = _dot(hn, w2t_ref[...]) + b2_ref[...]
    h2_ref[0] = h2

    @pl.when((b == 0) & (nb == 0))
    def _():
        stats_ref[...] = jnp.zeros((8, _H2), jnp.float32)

    stats_ref[0:1, :] += jnp.sum(h2, axis=0, keepdims=True)
    stats_ref[1:2, :] += jnp.sum(h2 * h2, axis=0, keepdims=True)


def _k4_body(h2_ref, stats2_ref, g2_ref, beta2_ref, out_ref):
    scale, shift = _bn_scale_shift(stats2_ref[...], g2_ref[...], beta2_ref[...], _H2)
    out_ref[0] = jnp.maximum(h2_ref[0] * scale + shift, 0.0)


def kernel(xyz1, xyz2, points1, points2, W1, b1, g1, beta1, W2, b2, g2, beta2):
    xyz2t = jnp.transpose(xyz2, (0, 2, 1))
    w1at = jnp.transpose(W1[:, :_C1])
    w1bt = jnp.transpose(W1[:, _C1:])
    w2t = jnp.transpose(W2)
    b1r = b1.reshape(1, _H1)
    b2r = b2.reshape(1, _H2)
    g1r = g1.reshape(1, _H1)
    beta1r = beta1.reshape(1, _H1)
    g2r = g2.reshape(1, _H2)
    beta2r = beta2.reshape(1, _H2)

    grid = (_B, _NB)

    gidx, wexp = pl.pallas_call(
        _k1_body,
        grid=grid,
        in_specs=[
            pl.BlockSpec((1, _BN, 3), lambda b, n: (b, n, 0)),
            pl.BlockSpec((1, 3, _M), lambda b, n: (b, 0, 0)),
        ],
        out_specs=[
            pl.BlockSpec((1, _BN, 3), lambda b, n: (b, n, 0)),
            pl.BlockSpec((1, _BN, 48), lambda b, n: (b, n, 0)),
        ],
        out_shape=[
            jax.ShapeDtypeStruct((_B, _N, 3), jnp.int32),
            jax.ShapeDtypeStruct((_B, _N, 48), jnp.float32),
        ],
        interpret=_interpret,
    )(xyz1, xyz2t)

    interp_flat = _sc_interp(points2.reshape(_NROWS, _C2),
                             gidx.reshape(_NPTS * 3),
                             wexp.reshape(_NPTS * 48))
    interp = interp_flat.reshape(_B, _N, _C2)

    h1, stats1 = pl.pallas_call(
        _k2_body,
        grid=grid,
        in_specs=[
            pl.BlockSpec((1, _BN, _C1), lambda b, n: (b, n, 0)),
            pl.BlockSpec((1, _BN, _C2), lambda b, n: (b, n, 0)),
            pl.BlockSpec((_C1, _H1), lambda b, n: (0, 0)),
            pl.BlockSpec((_C2, _H1), lambda b, n: (0, 0)),
            pl.BlockSpec((1, _H1), lambda b, n: (0, 0)),
        ],
        out_specs=[
            pl.BlockSpec((1, _BN, _H1), lambda b, n: (b, n, 0)),
            pl.BlockSpec((8, _H1), lambda b, n: (0, 0)),
        ],
        out_shape=[
            jax.ShapeDtypeStruct((_B, _N, _H1), jnp.float32),
            jax.ShapeDtypeStruct((8, _H1), jnp.float32),
        ],
        interpret=_interpret,
    )(points1, interp, w1at, w1bt, b1r)

    h2, stats2 = pl.pallas_call(
        _k3_body,
        grid=grid,
        in_specs=[
            pl.BlockSpec((1, _BN, _H1), lambda b, n: (b, n, 0)),
            pl.BlockSpec((8, _H1), lambda b, n: (0, 0)),
            pl.BlockSpec((1, _H1), lambda b, n: (0, 0)),
            pl.BlockSpec((1, _H1), lambda b, n: (0, 0)),
            pl.BlockSpec((_H1, _H2), lambda b, n: (0, 0)),
            pl.BlockSpec((1, _H2), lambda b, n: (0, 0)),
        ],
        out_specs=[
            pl.BlockSpec((1, _BN, _H2), lambda b, n: (b, n, 0)),
            pl.BlockSpec((8, _H2), lambda b, n: (0, 0)),
        ],
        out_shape=[
            jax.ShapeDtypeStruct((_B, _N, _H2), jnp.float32),
            jax.ShapeDtypeStruct((8, _H2), jnp.float32),
        ],
        interpret=_interpret,
    )(h1, stats1, g1r, beta1r, w2t, b2r)

    out = pl.pallas_call(
        _k4_body,
        grid=grid,
        in_specs=[
            pl.BlockSpec((1, _BN, _H2), lambda b, n: (b, n, 0)),
            pl.BlockSpec((8, _H2), lambda b, n: (0, 0)),
            pl.BlockSpec((1, _H2), lambda b, n: (0, 0)),
            pl.BlockSpec((1, _H2), lambda b, n: (0, 0)),
        ],
        out_specs=pl.BlockSpec((1, _BN, _H2), lambda b, n: (b, n, 0)),
        out_shape=jax.ShapeDtypeStruct((_B, _N, _H2), jnp.float32),
        interpret=_interpret,
    )(h2, stats2, g2r, beta2r)

    return out


# speedup vs baseline: 1.2199x; 1.0005x over previous
import functools

import jax
import jax.numpy as jnp
from jax import lax
from jax.experimental import pallas as pl
from jax.experimental.pallas import tpu as pltpu, tpu_sc as plsc

_B, _N, _M = 8, 4096, 1024
_C1, _C2 = 128, 256
_H1, _H2 = 256, 128
_BN = 512
_NB = _N // _BN

_NPTS = _B * _N
_NROWS = _B * _M
_NW = 32
_PPW = _NPTS // _NW
_CHUNK = 32
_NCH = _PPW // _CHUNK
_IDXC = 3 * _CHUNK
_WEXPC = _IDXC * 16

_interpret = False


def _dot(a, b):
    return jax.lax.dot_general(
        a, b, (((1,), (0,)), ((), ())),
        preferred_element_type=jnp.float32)


def _k1_body(xyz1_ref, xyz2t_ref, idx_ref, wexp_ref):
    b = pl.program_id(0)

    x1 = xyz1_ref[0]
    x2t = xyz2t_ref[0]

    mm = _dot(x1, x2t)
    s1 = x1[:, 0:1] * x1[:, 0:1] + x1[:, 1:2] * x1[:, 1:2] + x1[:, 2:3] * x1[:, 2:3]
    s2 = x2t[0:1, :] * x2t[0:1, :] + x2t[1:2, :] * x2t[1:2, :] + x2t[2:3, :] * x2t[2:3, :]
    d = -2.0 * mm
    d = d + s1
    d = d + s2

    iotaf = jax.lax.broadcasted_iota(jnp.int32, (_BN, _M), 1).astype(jnp.float32)
    big = jnp.float32(jnp.inf)
    recips = []
    cols = []
    for _ in range(3):
        m = jnp.min(d, axis=1, keepdims=True)
        colf = jnp.min(jnp.where(d == m, iotaf, jnp.float32(_M)),
                       axis=1, keepdims=True)
        mask = iotaf == colf
        d = jnp.where(mask, big, d)
        recips.append(1.0 / (m + 1e-8))
        cols.append(colf.astype(jnp.int32) + b * _M)
    norm = recips[0] + recips[1] + recips[2]

    idx_ref[0] = jnp.concatenate(cols, axis=1)
    ws = [jnp.broadcast_to(recips[k] / norm, (_BN, 16)) for k in range(3)]
    wexp_ref[0] = jnp.concatenate(ws, axis=1)


_sc_mesh = plsc.VectorSubcoreMesh(core_axis_name="c", subcore_axis_name="s")


@functools.partial(
    pl.kernel,
    out_type=jax.ShapeDtypeStruct((_NPTS * _C2,), jnp.float32),
    mesh=_sc_mesh,
    scratch_types=[
        pltpu.VMEM((3 * _PPW,), jnp.int32),
        pltpu.VMEM((_IDXC, _C2), jnp.float32),
        pltpu.VMEM((_IDXC, _C2), jnp.float32),
        pltpu.VMEM((_WEXPC,), jnp.float32),
        pltpu.VMEM((_WEXPC,), jnp.float32),
        pltpu.VMEM((_CHUNK * _C2,), jnp.float32),
        pltpu.SemaphoreType.DMA,
        pltpu.SemaphoreType.DMA,
        pltpu.SemaphoreType.DMA,
        pltpu.SemaphoreType.DMA,
    ],
)
def _sc_interp(p2_hbm, gidx_hbm, wexp_hbm, out_hbm,
               idx_all, rows0, rows1, w0, w1, out_v,
               sg0, sg1, sw0, sw1):
    wid = lax.axis_index("s") * 2 + lax.axis_index("c")
    base_i = wid * (_PPW * 3)

    pltpu.sync_copy(gidx_hbm.at[pl.ds(base_i, _PPW * 3)], idx_all)

    def start_chunk(c, rows_buf, w_buf, sg, sw):
        pltpu.async_copy(
            p2_hbm.at[idx_all.at[pl.ds(c * _IDXC, _IDXC)]], rows_buf, sg)
        pltpu.async_copy(
            wexp_hbm.at[pl.ds((base_i + c * _IDXC) * 16, _WEXPC)], w_buf, sw)

    def wait_chunk(rows_buf, w_buf, sg, sw):
        pltpu.make_async_copy(p2_hbm.at[pl.ds(0, _IDXC)], rows_buf, sg).wait()
        pltpu.make_async_copy(wexp_hbm.at[pl.ds(0, _WEXPC)], w_buf, sw).wait()

    def compute_chunk(c, rows_v, w_v):
        def point_body(half, _):
            for u in range(2):
                i = 2 * half + u
                pw0 = w_v[pl.ds((3 * i) * 16, 16)]
                pw1 = w_v[pl.ds((3 * i + 1) * 16, 16)]
                pw2 = w_v[pl.ds((3 * i + 2) * 16, 16)]
                for c16 in range(_C2 // 16):
                    o = c16 * 16
                    r0 = rows_v[3 * i, pl.ds(o, 16)]
                    r1 = rows_v[3 * i + 1, pl.ds(o, 16)]
                    r2 = rows_v[3 * i + 2, pl.ds(o, 16)]
                    acc = r0 * pw0
                    acc = acc + r1 * pw1
                    acc = acc + r2 * pw2
                    out_v[pl.ds(i * _C2 + o, 16)] = acc
            return 0

        lax.fori_loop(0, _CHUNK // 2, point_body, 0)
        dst = (wid * _PPW + c * _CHUNK) * _C2
        pltpu.sync_copy(out_v, out_hbm.at[pl.ds(dst, _CHUNK * _C2)])

    start_chunk(0, rows0, w0, sg0, sw0)

    def _ring_body(j, _):
        a = 2 * j
        b = a + 1
        start_chunk(b, rows1, w1, sg1, sw1)
        wait_chunk(rows0, w0, sg0, sw0)
        compute_chunk(a, rows0, w0)

        @pl.when(b + 1 < _NCH)
        def _():
            start_chunk(b + 1, rows0, w0, sg0, sw0)

        wait_chunk(rows1, w1, sg1, sw1)
        compute_chunk(b, rows1, w1)
        return 0

    lax.fori_loop(0, _NCH // 2, _ring_body, 0)


def _k2_body(p1_ref, interp_ref, w1at_ref, w1bt_ref, b1_ref, h1_ref, stats_ref):
    b = pl.program_id(0)
    nb = pl.program_id(1)
    h1 = _dot(p1_ref[0], w1at_ref[...]) + _dot(interp_ref[0], w1bt_ref[...])
    h1 = h1 + b1_ref[...]
    h1_ref[0] = h1

    @pl.when((b == 0) & (nb == 0))
    def _():
        stats_ref[...] = jnp.zeros((8, _H1), jnp.float32)

    stats_ref[0:1, :] += jnp.sum(h1, axis=0, keepdims=True)
    stats_ref[1:2, :] += jnp.sum(h1 * h1, axis=0, keepdims=True)


def _bn_scale_shift(stats, g, beta, h_dim):
    cnt = jnp.float32(_B * _N)
    mean = stats[0:1, :] / cnt
    var = stats[1:2, :] / cnt - mean * mean
    rstd = jax.lax.rsqrt(var + 1e-5)
    scale = rstd * g
    shift = beta - mean * scale
    return scale, shift


def _k3_body(h1_ref, stats1_ref, g1_ref, beta1_ref, w2t_ref, b2_ref,
             h2_ref, stats_ref):
    b = pl.program_id(0)
    nb = pl.program_id(1)
    scale, shift = _bn_scale_shift(stats1_ref[...], g1_ref[...], beta1_ref[...], _H1)
    hn = jnp.maximum(h1_ref[0] * scale + shift, 0.0)
    h2 = _dot(hn, w2t_ref[...]) + b2_ref[...]
    h2_ref[0] = h2

    @pl.when((b == 0) & (nb == 0))
    def _():
        stats_ref[...] = jnp.zeros((8, _H2), jnp.float32)

    stats_ref[0:1, :] += jnp.sum(h2, axis=0, keepdims=True)
    stats_ref[1:2, :] += jnp.sum(h2 * h2, axis=0, keepdims=True)


def _k4_body(h2_ref, stats2_ref, g2_ref, beta2_ref, out_ref):
    scale, shift = _bn_scale_shift(stats2_ref[...], g2_ref[...], beta2_ref[...], _H2)
    out_ref[0] = jnp.maximum(h2_ref[0] * scale + shift, 0.0)


def kernel(xyz1, xyz2, points1, points2, W1, b1, g1, beta1, W2, b2, g2, beta2):
    xyz2t = jnp.transpose(xyz2, (0, 2, 1))
    w1at = jnp.transpose(W1[:, :_C1])
    w1bt = jnp.transpose(W1[:, _C1:])
    w2t = jnp.transpose(W2)
    b1r = b1.reshape(1, _H1)
    b2r = b2.reshape(1, _H2)
    g1r = g1.reshape(1, _H1)
    beta1r = beta1.reshape(1, _H1)
    g2r = g2.reshape(1, _H2)
    beta2r = beta2.reshape(1, _H2)

    grid = (_B, _NB)

    gidx, wexp = pl.pallas_call(
        _k1_body,
        grid=grid,
        in_specs=[
            pl.BlockSpec((1, _BN, 3), lambda b, n: (b, n, 0)),
            pl.BlockSpec((1, 3, _M), lambda b, n: (b, 0, 0)),
        ],
        out_specs=[
            pl.BlockSpec((1, _BN, 3), lambda b, n: (b, n, 0)),
            pl.BlockSpec((1, _BN, 48), lambda b, n: (b, n, 0)),
        ],
        out_shape=[
            jax.ShapeDtypeStruct((_B, _N, 3), jnp.int32),
            jax.ShapeDtypeStruct((_B, _N, 48), jnp.float32),
        ],
        interpret=_interpret,
    )(xyz1, xyz2t)

    interp_flat = _sc_interp(points2.reshape(_NROWS, _C2),
                             gidx.reshape(_NPTS * 3),
                             wexp.reshape(_NPTS * 48))
    interp = interp_flat.reshape(_B, _N, _C2)

    h1, stats1 = pl.pallas_call(
        _k2_body,
        grid=grid,
        in_specs=[
            pl.BlockSpec((1, _BN, _C1), lambda b, n: (b, n, 0)),
            pl.BlockSpec((1, _BN, _C2), lambda b, n: (b, n, 0)),
            pl.BlockSpec((_C1, _H1), lambda b, n: (0, 0)),
            pl.BlockSpec((_C2, _H1), lambda b, n: (0, 0)),
            pl.BlockSpec((1, _H1), lambda b, n: (0, 0)),
        ],
        out_specs=[
            pl.BlockSpec((1, _BN, _H1), lambda b, n: (b, n, 0)),
            pl.BlockSpec((8, _H1), lambda b, n: (0, 0)),
        ],
        out_shape=[
            jax.ShapeDtypeStruct((_B, _N, _H1), jnp.float32),
            jax.ShapeDtypeStruct((8, _H1), jnp.float32),
        ],
        interpret=_interpret,
    )(points1, interp, w1at, w1bt, b1r)

    h2, stats2 = pl.pallas_call(
        _k3_body,
        grid=grid,
        in_specs=[
            pl.BlockSpec((1, _BN, _H1), lambda b, n: (b, n, 0)),
            pl.BlockSpec((8, _H1), lambda b, n: (0, 0)),
            pl.BlockSpec((1, _H1), lambda b, n: (0, 0)),
            pl.BlockSpec((1, _H1), lambda b, n: (0, 0)),
            pl.BlockSpec((_H1, _H2), lambda b, n: (0, 0)),
            pl.BlockSpec((1, _H2), lambda b, n: (0, 0)),
        ],
        out_specs=[
            pl.BlockSpec((1, _BN, _H2), lambda b, n: (b, n, 0)),
            pl.BlockSpec((8, _H2), lambda b, n: (0, 0)),
        ],
        out_shape=[
            jax.ShapeDtypeStruct((_B, _N, _H2), jnp.float32),
            jax.ShapeDtypeStruct((8, _H2), jnp.float32),
        ],
        interpret=_interpret,
    )(h1, stats1, g1r, beta1r, w2t, b2r)

    out = pl.pallas_call(
        _k4_body,
        grid=grid,
        in_specs=[
            pl.BlockSpec((1, _BN, _H2), lambda b, n: (b, n, 0)),
            pl.BlockSpec((8, _H2), lambda b, n: (0, 0)),
            pl.BlockSpec((1, _H2), lambda b, n: (0, 0)),
            pl.BlockSpec((1, _H2), lambda b, n: (0, 0)),
        ],
        out_specs=pl.BlockSpec((1, _BN, _H2), lambda b, n: (b, n, 0)),
        out_shape=jax.ShapeDtypeStruct((_B, _N, _H2), jnp.float32),
        interpret=_interpret,
    )(h2, stats2, g2r, beta2r)

    return out
